# baseline (speedup 1.0000x reference)
import functools

import jax
import jax.numpy as jnp
from jax import lax
from jax.experimental import pallas as pl
from jax.experimental.pallas import tpu as pltpu
from jax.experimental.pallas import tpu_sc as plsc

N = 16384
V = 1000000
DIM = 64
SEL = 2
OUT_W = DIM + SEL
PAD_W = 128

NC = 2
NS = 16
L = 16
NW = NC * NS

WIN = 256
NWIN_TOT = V // WIN
TAIL_BASE = NWIN_TOT * WIN
TAIL_W = V - TAIL_BASE
WPW = NWIN_TOT // NW

CAP = 736
DUMP = N
OUT_ROWS = N + 8

IDX_CHUNK = 2048


def _make_sc_gather():
    mesh = plsc.VectorSubcoreMesh(core_axis_name="c", subcore_axis_name="s")

    @functools.partial(
        pl.kernel,
        mesh=mesh,
        out_type=jax.ShapeDtypeStruct((OUT_ROWS, PAD_W), jnp.float32),
        compiler_params=pltpu.CompilerParams(needs_layout_passes=False),
        scratch_types=[
            pltpu.VMEM((IDX_CHUNK,), jnp.int32),
            pltpu.VMEM((CAP,), jnp.int32),
            pltpu.VMEM((CAP,), jnp.int32),
            pltpu.VMEM((CAP,), jnp.int32),
            pltpu.VMEM((64, WIN), jnp.float32),
            pltpu.VMEM((64, PAD_W), jnp.float32),
            pltpu.VMEM((CAP, PAD_W), jnp.float32),
            pltpu.SemaphoreType.DMA,
        ],
    )
    def k(vocab_hbm, tab_hbm, tail_hbm, out_hbm,
          idx_v, myv, myn, nlist, stage, tailst, rows, sem):
        cid = lax.axis_index("c")
        scid = lax.axis_index("s")
        wid = scid * NC + cid
        is_last = wid == NW - 1
        nwin = WPW + jnp.where(is_last, NWIN_TOT - NW * WPW, 0)
        lo = wid * (WPW * WIN)
        hi = jnp.where(is_last, V, lo + WPW * WIN)

        lanes = lax.iota(jnp.int32, L)
        zeros = jnp.zeros((L,), jnp.int32)

        def bin_outer(s, cnt):
            pltpu.sync_copy(vocab_hbm.at[pl.ds(s * IDX_CHUNK, IDX_CHUNK)],
                            idx_v)

            def bin_inner(c, cnt):
                v = idx_v[pl.ds(c * L, L)]
                m = (v >= lo) & (v < hi)
                slot = jnp.minimum(cnt, CAP - L)
                plsc.store_compressed(myv.at[pl.ds(slot, L)], v, mask=m)
                n = lanes + (s * IDX_CHUNK + c * L)
                plsc.store_compressed(myn.at[pl.ds(slot, L)], n, mask=m)
                return cnt + jnp.sum(m.astype(jnp.int32))

            return lax.fori_loop(0, IDX_CHUNK // L, bin_inner, cnt)

        cnt = lax.fori_loop(0, N // IDX_CHUNK, bin_outer, 0)
        nchunk = (cnt + L - 1) // L

        def dump_body(c, _):
            nlist[pl.ds(c * L, L)] = jnp.full((L,), DUMP, jnp.int32)
            return 0

        lax.fori_loop(0, CAP // L, dump_body, 0)

        def extract(st_ref, st_w, win_lo, win_w, c, _):
            gpos = lanes + c * L
            v = myv[pl.ds(c * L, L)]
            m = (v >= win_lo) & (v < win_lo + win_w) & (gpos < cnt)
            km = jnp.sum(m.astype(jnp.int32))

            @pl.when(km > 0)
            def _():
                n = myn[pl.ds(c * L, L)]
                cols = jnp.where(m, v - win_lo, 0)
                slotv = gpos
                plsc.store_scatter(nlist, [slotv], n, mask=m)
                for dd in range(DIM):
                    vals = plsc.load_gather(
                        st_ref, [zeros + dd, cols], mask=m)
                    plsc.store_scatter(
                        rows, [slotv, zeros + dd], vals, mask=m)
            return 0

        def win_body(j, _):
            win_lo = lo + j * WIN
            pltpu.sync_copy(
                tab_hbm.at[:, pl.ds(win_lo, WIN)], stage)
            lax.fori_loop(
                0, nchunk,
                functools.partial(extract, stage, WIN, win_lo, WIN), 0)
            return 0

        lax.fori_loop(0, nwin, win_body, 0)

        @pl.when(is_last)
        def _():
            pltpu.sync_copy(tail_hbm, tailst)
            lax.fori_loop(
                0, nchunk,
                functools.partial(extract, tailst, PAD_W, TAIL_BASE, TAIL_W),
                0)

        pltpu.async_copy(rows, out_hbm.at[nlist], sem).wait()

    return k


TC_ROWS = 1024


def _tc_finish(rows_ref, sid_ref, st_ref, o_ref):
    emb = rows_ref[:, :DIM]
    pick0 = sid_ref[...] == 0
    selrow = jnp.where(pick0, st_ref[0:1, :], st_ref[1:2, :])
    o_ref[...] = jnp.concatenate([emb, selrow], axis=1)


def _make_tc_finish():
    return pl.pallas_call(
        _tc_finish,
        grid=(N // TC_ROWS,),
        in_specs=[
            pl.BlockSpec((TC_ROWS, PAD_W), lambda i: (i, 0)),
            pl.BlockSpec((TC_ROWS, 1), lambda i: (i, 0)),
            pl.BlockSpec((2, SEL), lambda i: (0, 0)),
        ],
        out_specs=pl.BlockSpec((TC_ROWS, OUT_W), lambda i: (i, 0)),
        out_shape=jax.ShapeDtypeStruct((N, OUT_W), jnp.float32),
    )


@jax.jit
def kernel(vocab_ids, selector_ids, table, selector_table):
    vocab_ids = vocab_ids.astype(jnp.int32)
    selector_ids = selector_ids.astype(jnp.int32)
    tab_t = table.T
    tail = jnp.pad(tab_t[:, TAIL_BASE:], ((0, 0), (0, PAD_W - TAIL_W)))
    rows = _make_sc_gather()(vocab_ids, tab_t, tail)
    return _make_tc_finish()(rows,
                             selector_ids.reshape(N, 1),
                             selector_table.astype(jnp.float32))


# speedup vs baseline: 1.4464x; 1.4464x over previous
import functools

import jax
import jax.numpy as jnp
from jax import lax
from jax.experimental import pallas as pl
from jax.experimental.pallas import tpu as pltpu
from jax.experimental.pallas import tpu_sc as plsc

N = 16384
V = 1000000
DIM = 64
SEL = 2
OUT_W = DIM + SEL
PAD_W = 128

NC = 2
NS = 16
L = 16
NW = NC * NS

WIN = 256
NWIN_TOT = V // WIN
TAIL_BASE = NWIN_TOT * WIN
TAIL_W = V - TAIL_BASE
WPW = NWIN_TOT // NW

CAP = 672
DUMP = N
OUT_ROWS = N + 8

IDX_CHUNK = 1024


def _make_sc_gather():
    mesh = plsc.VectorSubcoreMesh(core_axis_name="c", subcore_axis_name="s")

    @functools.partial(
        pl.kernel,
        mesh=mesh,
        out_type=jax.ShapeDtypeStruct((OUT_ROWS, PAD_W), jnp.float32),
        compiler_params=pltpu.CompilerParams(needs_layout_passes=False),
        scratch_types=[
            pltpu.VMEM((IDX_CHUNK,), jnp.int32),
            pltpu.VMEM((CAP,), jnp.int32),
            pltpu.VMEM((CAP,), jnp.int32),
            pltpu.VMEM((CAP,), jnp.int32),
            pltpu.VMEM((2, 64, WIN), jnp.float32),
            pltpu.VMEM((64, PAD_W), jnp.float32),
            pltpu.VMEM((CAP, PAD_W), jnp.float32),
            pltpu.SemaphoreType.DMA,
            pltpu.SemaphoreType.DMA,
            pltpu.SemaphoreType.DMA,
        ],
    )
    def k(vocab_hbm, tab_hbm, tail_hbm, out_hbm,
          idx_v, myv, myn, nlist, stage, tailst, rows, sem0, sem1, sem2):
        cid = lax.axis_index("c")
        scid = lax.axis_index("s")
        wid = scid * NC + cid
        is_last = wid == NW - 1
        nwin = WPW + jnp.where(is_last, NWIN_TOT - NW * WPW, 0)
        lo = wid * (WPW * WIN)
        hi = jnp.where(is_last, V, lo + WPW * WIN)

        lanes = lax.iota(jnp.int32, L)
        zeros = jnp.zeros((L,), jnp.int32)

        def bin_outer(s, cnt):
            pltpu.sync_copy(vocab_hbm.at[pl.ds(s * IDX_CHUNK, IDX_CHUNK)],
                            idx_v)

            def bin_inner(c, cnt):
                v = idx_v[pl.ds(c * L, L)]
                m = (v >= lo) & (v < hi)
                slot = jnp.minimum(cnt, CAP - L)
                plsc.store_compressed(myv.at[pl.ds(slot, L)], v, mask=m)
                n = lanes + (s * IDX_CHUNK + c * L)
                plsc.store_compressed(myn.at[pl.ds(slot, L)], n, mask=m)
                return cnt + jnp.sum(m.astype(jnp.int32))

            return lax.fori_loop(0, IDX_CHUNK // L, bin_inner, cnt)

        cnt = lax.fori_loop(0, N // IDX_CHUNK, bin_outer, 0)
        nchunk = (cnt + L - 1) // L

        def dump_body(c, _):
            nlist[pl.ds(c * L, L)] = jnp.full((L,), DUMP, jnp.int32)
            return 0

        lax.fori_loop(0, CAP // L, dump_body, 0)

        def extract(st_ref, st_w, win_lo, win_w, c, _):
            gpos = lanes + c * L
            v = myv[pl.ds(c * L, L)]
            m = (v >= win_lo) & (v < win_lo + win_w) & (gpos < cnt)
            km = jnp.sum(m.astype(jnp.int32))

            @pl.when(km > 0)
            def _():
                n = myn[pl.ds(c * L, L)]
                cols = jnp.where(m, v - win_lo, 0)
                slotv = gpos
                plsc.store_scatter(nlist, [slotv], n, mask=m)
                for dd in range(DIM):
                    vals = plsc.load_gather(
                        st_ref, [zeros + dd, cols], mask=m)
                    plsc.store_scatter(
                        rows, [slotv, zeros + dd], vals, mask=m)
            return 0

        bufs = (stage.at[0], stage.at[1])
        sems = (sem0, sem1)

        def start(g, b):
            @pl.when(g < nwin)
            def _():
                pltpu.async_copy(
                    tab_hbm.at[:, pl.ds(lo + g * WIN, WIN)], bufs[b], sems[b])

        def wait_extract(g, b):
            @pl.when(g < nwin)
            def _():
                pltpu.make_async_copy(
                    tab_hbm.at[:, pl.ds(lo + g * WIN, WIN)],
                    bufs[b], sems[b]).wait()
                lax.fori_loop(
                    0, nchunk,
                    functools.partial(extract, bufs[b], WIN, lo + g * WIN,
                                      WIN), 0)

        start(0, 0)
        start(1, 1)

        def pair_body(p, _):
            g = 2 * p
            wait_extract(g, 0)
            start(g + 2, 0)
            wait_extract(g + 1, 1)
            start(g + 3, 1)
            return 0

        max_pairs = (WPW + (NWIN_TOT - NW * WPW) + 1) // 2
        lax.fori_loop(0, max_pairs, pair_body, 0)

        @pl.when(is_last)
        def _():
            tail_buf = tailst
            pltpu.sync_copy(tail_hbm, tail_buf)
            lax.fori_loop(
                0, nchunk,
                functools.partial(extract, tail_buf, PAD_W, TAIL_BASE,
                                  TAIL_W), 0)

        pltpu.async_copy(rows, out_hbm.at[nlist], sem2).wait()

    return k


TC_ROWS = 1024


def _tc_finish(rows_ref, sid_ref, st_ref, o_ref):
    emb = rows_ref[:, :DIM]
    pick0 = sid_ref[...] == 0
    selrow = jnp.where(pick0, st_ref[0:1, :], st_ref[1:2, :])
    o_ref[...] = jnp.concatenate([emb, selrow], axis=1)


def _make_tc_finish():
    return pl.pallas_call(
        _tc_finish,
        grid=(N // TC_ROWS,),
        in_specs=[
            pl.BlockSpec((TC_ROWS, PAD_W), lambda i: (i, 0)),
            pl.BlockSpec((TC_ROWS, 1), lambda i: (i, 0)),
            pl.BlockSpec((2, SEL), lambda i: (0, 0)),
        ],
        out_specs=pl.BlockSpec((TC_ROWS, OUT_W), lambda i: (i, 0)),
        out_shape=jax.ShapeDtypeStruct((N, OUT_W), jnp.float32),
    )


@jax.jit
def kernel(vocab_ids, selector_ids, table, selector_table):
    vocab_ids = vocab_ids.astype(jnp.int32)
    selector_ids = selector_ids.astype(jnp.int32)
    tab_t = table.T
    tail = jnp.pad(tab_t[:, TAIL_BASE:], ((0, 0), (0, PAD_W - TAIL_W)))
    rows = _make_sc_gather()(vocab_ids, tab_t, tail)
    return _make_tc_finish()(rows,
                             selector_ids.reshape(N, 1),
                             selector_table.astype(jnp.float32))
